# Initial kernel scaffold; baseline (speedup 1.0000x reference)
import jax
import jax.numpy as jnp
from jax.experimental import pallas as pl


def kernel(ents_w, rels_w, heads, rels, tails, heads_bad, rels_bad, tails_bad):
    raise NotImplementedError("write your pallas kernel here")


# speedup vs baseline: 1.0185x; 1.0185x over previous
import functools

import jax
import jax.numpy as jnp
from jax.experimental import pallas as pl
from jax.experimental.pallas import tpu as pltpu


def _reg_cast_kernel(w_ref, wb_ref, part_ref):
    w = w_ref[...]
    wb_ref[...] = w.astype(jnp.bfloat16)
    ss = jnp.sum(w * w, axis=1, keepdims=True)
    reg = jnp.maximum(jnp.sqrt(ss) - 1.0, 0.0)
    part_ref[...] = jnp.full((1, 1, 128), jnp.sum(reg), dtype=jnp.float32)


def _loss_kernel(gh_ref, gt_ref, bh_ref, bt_ref, gr_ref, br_ref, part_ref,
                 *, margin, l2reg):
    gd = (gh_ref[...].astype(jnp.float32) + gr_ref[...].astype(jnp.float32)
          - gt_ref[...].astype(jnp.float32))
    gold = jnp.sqrt(jnp.sum(gd * gd, axis=1, keepdims=True))
    bd = (bh_ref[...].astype(jnp.float32) + br_ref[...].astype(jnp.float32)
          - bt_ref[...].astype(jnp.float32))
    corrupt = jnp.sqrt(jnp.sum(bd * bd, axis=1, keepdims=True))
    contrib = jnp.maximum(margin + gold - corrupt, 0.0) + l2reg * gold
    part_ref[...] = jnp.full((1, 1, 128), jnp.sum(contrib), dtype=jnp.float32)


def kernel(ents_w, rels_w, heads, rels, tails, heads_bad, rels_bad, tails_bad):
    margin, l2reg = 1.0, 0.1
    nEnts, dim = ents_w.shape
    ents_w = ents_w.astype(jnp.float32)
    B = int(heads.shape[0])

    tile = 8192
    while nEnts % tile:
        tile //= 2
    n_tiles = nEnts // tile

    ents_bf, reg_part = pl.pallas_call(
        _reg_cast_kernel,
        out_shape=(jax.ShapeDtypeStruct((nEnts, dim), jnp.bfloat16),
                   jax.ShapeDtypeStruct((n_tiles, 1, 128), jnp.float32)),
        grid=(n_tiles,),
        in_specs=[pl.BlockSpec((tile, dim), lambda i: (i, 0))],
        out_specs=(pl.BlockSpec((tile, dim), lambda i: (i, 0)),
                   pl.BlockSpec((1, 1, 128), lambda i: (i, 0, 0))),
        compiler_params=pltpu.CompilerParams(
            dimension_semantics=("parallel",)),
    )(ents_w)

    rels_bf = rels_w.astype(jnp.bfloat16)

    idx = [jnp.asarray(a, dtype=jnp.int32)
           for a in (heads, tails, heads_bad, tails_bad, rels, rels_bad)]
    eslab = jnp.take(ents_bf, jnp.concatenate(idx[:4]), axis=0)
    rslab = jnp.take(rels_bf, jnp.concatenate(idx[4:]), axis=0)

    tb = 2048
    while B % tb:
        tb //= 2
    n_btiles = B // tb
    off = B // tb

    espec = [pl.BlockSpec((tb, dim), functools.partial(
        lambda k, i: (i + k * off, 0), k)) for k in range(4)]
    rspec = [pl.BlockSpec((tb, dim), functools.partial(
        lambda k, i: (i + k * off, 0), k)) for k in range(2)]

    loss_part = pl.pallas_call(
        functools.partial(_loss_kernel, margin=margin, l2reg=l2reg),
        out_shape=jax.ShapeDtypeStruct((n_btiles, 1, 128), jnp.float32),
        grid=(n_btiles,),
        in_specs=espec + rspec,
        out_specs=pl.BlockSpec((1, 1, 128), lambda i: (i, 0, 0)),
        compiler_params=pltpu.CompilerParams(
            dimension_semantics=("parallel",)),
    )(eslab, eslab, eslab, eslab, rslab, rslab)

    return jnp.sum(loss_part[:, 0, 0]) + l2reg * jnp.sum(reg_part[:, 0, 0])


# speedup vs baseline: 1.7666x; 1.7345x over previous
import functools

import jax
import jax.numpy as jnp
from jax.experimental import pallas as pl
from jax.experimental.pallas import tpu as pltpu


def _reg_cast_kernel(w_ref, wb_ref, part_ref):
    w = w_ref[...]
    wb_ref[...] = w.astype(jnp.bfloat16)
    ss = jnp.sum(w * w, axis=1, keepdims=True)
    reg = jnp.maximum(jnp.sqrt(ss) - 1.0, 0.0)
    part_ref[...] = jnp.full((1, 1, 128), jnp.sum(reg), dtype=jnp.float32)


def _loss_kernel(gh_ref, gt_ref, bh_ref, bt_ref, rw_ref, gi_ref, bi_ref,
                 part_ref, *, margin, l2reg, n_rels):
    tb = gh_ref.shape[0]
    rw = rw_ref[...]
    lanes = jax.lax.broadcasted_iota(jnp.int32, (tb, n_rels), 1)

    g_oh = (lanes == gi_ref[0]).astype(jnp.bfloat16)
    gr = jnp.dot(g_oh, rw, preferred_element_type=jnp.float32)
    gd = (gh_ref[...].astype(jnp.float32) - gt_ref[...].astype(jnp.float32)
          + gr)
    gold = jnp.sqrt(jnp.sum(gd * gd, axis=1, keepdims=True))

    b_oh = (lanes == bi_ref[0]).astype(jnp.bfloat16)
    br = jnp.dot(b_oh, rw, preferred_element_type=jnp.float32)
    bd = (bh_ref[...].astype(jnp.float32) - bt_ref[...].astype(jnp.float32)
          + br)
    corrupt = jnp.sqrt(jnp.sum(bd * bd, axis=1, keepdims=True))

    contrib = jnp.maximum(margin + gold - corrupt, 0.0) + l2reg * gold
    part_ref[...] = jnp.full((1, 1, 128), jnp.sum(contrib), dtype=jnp.float32)


def kernel(ents_w, rels_w, heads, rels, tails, heads_bad, rels_bad, tails_bad):
    margin, l2reg = 1.0, 0.1
    nEnts, dim = ents_w.shape
    nRels = rels_w.shape[0]
    ents_w = ents_w.astype(jnp.float32)
    B = int(heads.shape[0])

    tile = 8192
    while nEnts % tile:
        tile //= 2
    n_tiles = nEnts // tile

    ents_bf, reg_part = pl.pallas_call(
        _reg_cast_kernel,
        out_shape=(jax.ShapeDtypeStruct((nEnts, dim), jnp.bfloat16),
                   jax.ShapeDtypeStruct((n_tiles, 1, 128), jnp.float32)),
        grid=(n_tiles,),
        in_specs=[pl.BlockSpec((tile, dim), lambda i: (i, 0))],
        out_specs=(pl.BlockSpec((tile, dim), lambda i: (i, 0)),
                   pl.BlockSpec((1, 1, 128), lambda i: (i, 0, 0))),
        compiler_params=pltpu.CompilerParams(
            dimension_semantics=("parallel",)),
    )(ents_w)

    rels_bf = rels_w.astype(jnp.bfloat16)

    eidx = jnp.concatenate([jnp.asarray(a, dtype=jnp.int32)
                            for a in (heads, tails, heads_bad, tails_bad)])
    eslab = jnp.take(ents_bf, eidx, axis=0)

    tb = 2048
    while B % tb:
        tb //= 2
    n_btiles = B // tb
    off = B // tb

    gi = jnp.asarray(rels, dtype=jnp.int32).reshape(n_btiles, tb, 1)
    bi = jnp.asarray(rels_bad, dtype=jnp.int32).reshape(n_btiles, tb, 1)

    espec = [pl.BlockSpec((tb, dim), functools.partial(
        lambda k, i: (i + k * off, 0), k)) for k in range(4)]
    rw_spec = pl.BlockSpec((nRels, dim), lambda i: (0, 0))
    idx_spec = pl.BlockSpec((1, tb, 1), lambda i: (i, 0, 0))

    loss_part = pl.pallas_call(
        functools.partial(_loss_kernel, margin=margin, l2reg=l2reg,
                          n_rels=nRels),
        out_shape=jax.ShapeDtypeStruct((n_btiles, 1, 128), jnp.float32),
        grid=(n_btiles,),
        in_specs=espec + [rw_spec, idx_spec, idx_spec],
        out_specs=pl.BlockSpec((1, 1, 128), lambda i: (i, 0, 0)),
        compiler_params=pltpu.CompilerParams(
            dimension_semantics=("parallel",)),
    )(eslab, eslab, eslab, eslab, rels_bf, gi, bi)

    return jnp.sum(loss_part[:, 0, 0]) + l2reg * jnp.sum(reg_part[:, 0, 0])


# speedup vs baseline: 1.9124x; 1.0825x over previous
import functools

import jax
import jax.numpy as jnp
from jax.experimental import pallas as pl
from jax.experimental.pallas import tpu as pltpu


def _reg_kernel(w_ref, part_ref):
    w = w_ref[...]
    ss = jnp.sum(w * w, axis=1, keepdims=True)
    reg = jnp.maximum(jnp.sqrt(ss) - 1.0, 0.0)
    part_ref[...] = jnp.full((1, 1, 128), jnp.sum(reg), dtype=jnp.float32)


def _loss_kernel(gh_ref, gt_ref, bh_ref, bt_ref, rw_ref, gi_ref, bi_ref,
                 part_ref, *, margin, l2reg, n_rels):
    tb = gh_ref.shape[0]
    rw = rw_ref[...]
    lanes = jax.lax.broadcasted_iota(jnp.int32, (tb, n_rels), 1)

    g_oh = (lanes == gi_ref[0]).astype(jnp.float32)
    gr = jnp.dot(g_oh, rw, preferred_element_type=jnp.float32)
    gd = gh_ref[...] - gt_ref[...] + gr
    gold = jnp.sqrt(jnp.sum(gd * gd, axis=1, keepdims=True))

    b_oh = (lanes == bi_ref[0]).astype(jnp.float32)
    br = jnp.dot(b_oh, rw, preferred_element_type=jnp.float32)
    bd = bh_ref[...] - bt_ref[...] + br
    corrupt = jnp.sqrt(jnp.sum(bd * bd, axis=1, keepdims=True))

    contrib = jnp.maximum(margin + gold - corrupt, 0.0) + l2reg * gold
    part_ref[...] = jnp.full((1, 1, 128), jnp.sum(contrib), dtype=jnp.float32)


def kernel(ents_w, rels_w, heads, rels, tails, heads_bad, rels_bad, tails_bad):
    margin, l2reg = 1.0, 0.1
    nEnts, dim = ents_w.shape
    nRels = rels_w.shape[0]
    ents_w = ents_w.astype(jnp.float32)
    B = int(heads.shape[0])

    tile = 8192
    while nEnts % tile:
        tile //= 2
    n_tiles = nEnts // tile

    reg_part = pl.pallas_call(
        _reg_kernel,
        out_shape=jax.ShapeDtypeStruct((n_tiles, 1, 128), jnp.float32),
        grid=(n_tiles,),
        in_specs=[pl.BlockSpec((tile, dim), lambda i: (i, 0))],
        out_specs=pl.BlockSpec((1, 1, 128), lambda i: (i, 0, 0)),
        compiler_params=pltpu.CompilerParams(
            dimension_semantics=("parallel",)),
    )(ents_w)

    rels_f = rels_w.astype(jnp.float32)

    eidx = jnp.concatenate([jnp.asarray(a, dtype=jnp.int32)
                            for a in (heads, tails, heads_bad, tails_bad)])
    eslab = jnp.take(ents_w, eidx, axis=0)

    tb = 2048
    while B % tb:
        tb //= 2
    n_btiles = B // tb
    off = B // tb

    gi = jnp.asarray(rels, dtype=jnp.int32).reshape(n_btiles, tb, 1)
    bi = jnp.asarray(rels_bad, dtype=jnp.int32).reshape(n_btiles, tb, 1)

    espec = [pl.BlockSpec((tb, dim), functools.partial(
        lambda k, i: (i + k * off, 0), k)) for k in range(4)]
    rw_spec = pl.BlockSpec((nRels, dim), lambda i: (0, 0))
    idx_spec = pl.BlockSpec((1, tb, 1), lambda i: (i, 0, 0))

    loss_part = pl.pallas_call(
        functools.partial(_loss_kernel, margin=margin, l2reg=l2reg,
                          n_rels=nRels),
        out_shape=jax.ShapeDtypeStruct((n_btiles, 1, 128), jnp.float32),
        grid=(n_btiles,),
        in_specs=espec + [rw_spec, idx_spec, idx_spec],
        out_specs=pl.BlockSpec((1, 1, 128), lambda i: (i, 0, 0)),
        compiler_params=pltpu.CompilerParams(
            dimension_semantics=("parallel",)),
    )(eslab, eslab, eslab, eslab, rels_f, gi, bi)

    return jnp.sum(loss_part[:, 0, 0]) + l2reg * jnp.sum(reg_part[:, 0, 0])


# speedup vs baseline: 3.6832x; 1.9260x over previous
import functools

import jax
import jax.numpy as jnp
from jax.experimental import pallas as pl
from jax.experimental.pallas import tpu as pltpu


def _reg_kernel(w_ref, part_ref):
    w = w_ref[...]
    ss = jnp.sum(w * w, axis=1, keepdims=True)
    reg = jnp.maximum(jnp.sqrt(ss) - 1.0, 0.0)
    part_ref[...] = jnp.full((1, 1, 128), jnp.sum(reg), dtype=jnp.float32)


def _loss_kernel(gh_ref, gt_ref, bh_ref, bt_ref, rw_ref, gi_ref, bi_ref,
                 part_ref, *, margin, l2reg, n_rels):
    tb = gh_ref.shape[0]
    rw = rw_ref[...]
    rows = jax.lax.broadcasted_iota(jnp.int32, (n_rels, tb), 0)
    dn = (((0,), (0,)), ((), ()))

    g_oht = (rows == gi_ref[0]).astype(jnp.float32)
    gr = jax.lax.dot_general(g_oht, rw, dn,
                             preferred_element_type=jnp.float32)
    gd = gh_ref[...] - gt_ref[...] + gr
    gold = jnp.sqrt(jnp.sum(gd * gd, axis=1, keepdims=True))

    b_oht = (rows == bi_ref[0]).astype(jnp.float32)
    br = jax.lax.dot_general(b_oht, rw, dn,
                             preferred_element_type=jnp.float32)
    bd = bh_ref[...] - bt_ref[...] + br
    corrupt = jnp.sqrt(jnp.sum(bd * bd, axis=1, keepdims=True))

    contrib = jnp.maximum(margin + gold - corrupt, 0.0) + l2reg * gold
    part_ref[...] = jnp.full((1, 1, 128), jnp.sum(contrib), dtype=jnp.float32)


def kernel(ents_w, rels_w, heads, rels, tails, heads_bad, rels_bad, tails_bad):
    margin, l2reg = 1.0, 0.1
    nEnts, dim = ents_w.shape
    nRels = rels_w.shape[0]
    ents_w = ents_w.astype(jnp.float32)
    B = int(heads.shape[0])

    tile = 8192
    while nEnts % tile:
        tile //= 2
    n_tiles = nEnts // tile

    reg_part = pl.pallas_call(
        _reg_kernel,
        out_shape=jax.ShapeDtypeStruct((n_tiles, 1, 128), jnp.float32),
        grid=(n_tiles,),
        in_specs=[pl.BlockSpec((tile, dim), lambda i: (i, 0))],
        out_specs=pl.BlockSpec((1, 1, 128), lambda i: (i, 0, 0)),
        compiler_params=pltpu.CompilerParams(
            dimension_semantics=("parallel",)),
    )(ents_w)

    rels_f = rels_w.astype(jnp.float32)

    eidx = jnp.concatenate([jnp.asarray(a, dtype=jnp.int32)
                            for a in (heads, tails, heads_bad, tails_bad)])
    eslab = ents_w.at[eidx].get(mode="promise_in_bounds")

    tb = 2048
    while B % tb:
        tb //= 2
    n_btiles = B // tb
    off = B // tb

    gi = jnp.asarray(rels, dtype=jnp.int32).reshape(n_btiles, 1, tb)
    bi = jnp.asarray(rels_bad, dtype=jnp.int32).reshape(n_btiles, 1, tb)

    espec = [pl.BlockSpec((tb, dim), functools.partial(
        lambda k, i: (i + k * off, 0), k)) for k in range(4)]
    rw_spec = pl.BlockSpec((nRels, dim), lambda i: (0, 0))
    idx_spec = pl.BlockSpec((1, 1, tb), lambda i: (i, 0, 0))

    loss_part = pl.pallas_call(
        functools.partial(_loss_kernel, margin=margin, l2reg=l2reg,
                          n_rels=nRels),
        out_shape=jax.ShapeDtypeStruct((n_btiles, 1, 128), jnp.float32),
        grid=(n_btiles,),
        in_specs=espec + [rw_spec, idx_spec, idx_spec],
        out_specs=pl.BlockSpec((1, 1, 128), lambda i: (i, 0, 0)),
        compiler_params=pltpu.CompilerParams(
            dimension_semantics=("parallel",)),
    )(eslab, eslab, eslab, eslab, rels_f, gi, bi)

    return jnp.sum(loss_part[:, 0, 0]) + l2reg * jnp.sum(reg_part[:, 0, 0])


# speedup vs baseline: 3.9444x; 1.0709x over previous
import functools

import jax
import jax.numpy as jnp
from jax.experimental import pallas as pl
from jax.experimental.pallas import tpu as pltpu

_DN_T = (((1,), (1,)), ((), ()))
_DN_0 = (((0,), (0,)), ((), ()))


def _reg_kernel(w_ref, part_ref):
    w = w_ref[...]
    ones = jnp.ones((1, w.shape[1]), jnp.float32)
    ss = jax.lax.dot_general(ones, w * w, _DN_T,
                             preferred_element_type=jnp.float32)
    reg = jnp.maximum(jnp.sqrt(ss) - 1.0, 0.0)
    part_ref[...] = jnp.full((1, 1, 128), jnp.sum(reg), dtype=jnp.float32)


def _loss_kernel(gh_ref, gt_ref, bh_ref, bt_ref, rw_ref, gi_ref, bi_ref,
                 part_ref, *, margin, l2reg, n_rels):
    tb = gh_ref.shape[0]
    dim = gh_ref.shape[1]
    rw = rw_ref[...]
    rows = jax.lax.broadcasted_iota(jnp.int16, (n_rels, tb), 0)
    ones = jnp.ones((1, dim), jnp.float32)

    one_b = jnp.bfloat16(1.0)
    zero_b = jnp.bfloat16(0.0)
    g_oht = jnp.where(rows == gi_ref[0], one_b, zero_b)
    gr = jax.lax.dot_general(g_oht, rw, _DN_0,
                             preferred_element_type=jnp.float32)
    gd = gh_ref[...] - gt_ref[...] + gr
    ssg = jax.lax.dot_general(ones, gd * gd, _DN_T,
                              preferred_element_type=jnp.float32)
    gold = jnp.sqrt(ssg)

    b_oht = jnp.where(rows == bi_ref[0], one_b, zero_b)
    br = jax.lax.dot_general(b_oht, rw, _DN_0,
                             preferred_element_type=jnp.float32)
    bd = bh_ref[...] - bt_ref[...] + br
    ssb = jax.lax.dot_general(ones, bd * bd, _DN_T,
                              preferred_element_type=jnp.float32)
    corrupt = jnp.sqrt(ssb)

    contrib = jnp.maximum(margin + gold - corrupt, 0.0) + l2reg * gold
    part_ref[...] = jnp.full((1, 1, 128), jnp.sum(contrib), dtype=jnp.float32)


def kernel(ents_w, rels_w, heads, rels, tails, heads_bad, rels_bad, tails_bad):
    margin, l2reg = 1.0, 0.1
    nEnts, dim = ents_w.shape
    nRels = rels_w.shape[0]
    ents_w = ents_w.astype(jnp.float32)
    B = int(heads.shape[0])

    tile = 8192
    while nEnts % tile:
        tile //= 2
    n_tiles = nEnts // tile

    reg_part = pl.pallas_call(
        _reg_kernel,
        out_shape=jax.ShapeDtypeStruct((n_tiles, 1, 128), jnp.float32),
        grid=(n_tiles,),
        in_specs=[pl.BlockSpec((tile, dim), lambda i: (i, 0))],
        out_specs=pl.BlockSpec((1, 1, 128), lambda i: (i, 0, 0)),
        compiler_params=pltpu.CompilerParams(
            dimension_semantics=("parallel",)),
    )(ents_w)

    rels_bf = rels_w.astype(jnp.bfloat16)

    eidx = jnp.concatenate([jnp.asarray(a, dtype=jnp.int32)
                            for a in (heads, tails, heads_bad, tails_bad)])
    eslab = ents_w.at[eidx].get(mode="promise_in_bounds")

    tb = 2048
    while B % tb:
        tb //= 2
    n_btiles = B // tb
    off = B // tb

    gi = jnp.asarray(rels, dtype=jnp.int16).reshape(n_btiles, 1, tb)
    bi = jnp.asarray(rels_bad, dtype=jnp.int16).reshape(n_btiles, 1, tb)

    espec = [pl.BlockSpec((tb, dim), functools.partial(
        lambda k, i: (i + k * off, 0), k)) for k in range(4)]
    rw_spec = pl.BlockSpec((nRels, dim), lambda i: (0, 0))
    idx_spec = pl.BlockSpec((1, 1, tb), lambda i: (i, 0, 0))

    loss_part = pl.pallas_call(
        functools.partial(_loss_kernel, margin=margin, l2reg=l2reg,
                          n_rels=nRels),
        out_shape=jax.ShapeDtypeStruct((n_btiles, 1, 128), jnp.float32),
        grid=(n_btiles,),
        in_specs=espec + [rw_spec, idx_spec, idx_spec],
        out_specs=pl.BlockSpec((1, 1, 128), lambda i: (i, 0, 0)),
        compiler_params=pltpu.CompilerParams(
            dimension_semantics=("parallel",)),
    )(eslab, eslab, eslab, eslab, rels_bf, gi, bi)

    return jnp.sum(loss_part[:, 0, 0]) + l2reg * jnp.sum(reg_part[:, 0, 0])
